# baseline (speedup 1.0000x reference)
import functools

import jax
import jax.numpy as jnp
from jax import lax
from jax.experimental import pallas as pl
from jax.experimental.pallas import tpu as pltpu
from jax.experimental.pallas import tpu_sc as plsc

_D = 128
_FEW = 64
_NEIGH = 200
_NPAD = 208
_NSYM = 100000
_QROWS = 8192
_NW = 32
_QPW = _QROWS // _NW
_SEGS = 2 * _FEW
_SPW = _SEGS // _NW


def _sc_support(sidx, table):
    mesh = plsc.VectorSubcoreMesh(core_axis_name="c", subcore_axis_name="s")

    nrows_w = _SPW * _NPAD

    @functools.partial(
        pl.kernel,
        out_type=jax.ShapeDtypeStruct((_SEGS, _D), jnp.float32),
        mesh=mesh,
        scratch_types=[
            pltpu.VMEM((nrows_w,), jnp.int32),
            pltpu.VMEM((nrows_w, _D), jnp.float32),
            pltpu.VMEM((_SPW, _D), jnp.float32),
            pltpu.SemaphoreType.DMA,
        ],
    )
    def k(sidx_hbm, table_hbm, ssum_hbm, sidx_v, srows_v, acc_v, sem):
        wid = lax.axis_index("s") * 2 + lax.axis_index("c")
        sbase = wid * _SPW
        pltpu.sync_copy(sidx_hbm.at[pl.ds(wid * nrows_w, nrows_w)], sidx_v)
        s_cp = []
        for ch in range(nrows_w // 128):
            s_cp.append(pltpu.async_copy(
                table_hbm.at[sidx_v.at[pl.ds(ch * 128, 128)]],
                srows_v.at[pl.ds(ch * 128, 128)], sem))
        tail = nrows_w - (nrows_w // 128) * 128
        if tail:
            s_cp.append(pltpu.async_copy(
                table_hbm.at[sidx_v.at[pl.ds(nrows_w - tail, tail)]],
                srows_v.at[pl.ds(nrows_w - tail, tail)], sem))
        for cp in s_cp:
            cp.wait()

        def accumulate(t):
            base = t * _NPAD

            def body(r, accs):
                new = accs
                for rr in range(4):
                    row = base + r * 4 + rr
                    new = tuple(a + srows_v[row, pl.ds(kk * 16, 16)]
                                for kk, a in enumerate(new))
                return new

            accs = lax.fori_loop(
                0, _NPAD // 4, body,
                tuple(jnp.zeros((16,), jnp.float32) for _ in range(_D // 16)))
            for kk in range(_D // 16):
                acc_v[t, pl.ds(kk * 16, 16)] = accs[kk]

        for t in range(_SPW):
            accumulate(t)
        pltpu.sync_copy(acc_v, ssum_hbm.at[pl.ds(sbase, _SPW)])

    return k(sidx, table)


def _sc_qgather(qidx, table, nrows):
    per_w = nrows // _NW
    mesh = plsc.VectorSubcoreMesh(core_axis_name="c", subcore_axis_name="s")

    @functools.partial(
        pl.kernel,
        out_type=jax.ShapeDtypeStruct((nrows, _D), jnp.float32),
        mesh=mesh,
        scratch_types=[
            pltpu.VMEM((per_w,), jnp.int32),
            pltpu.VMEM((per_w, _D), jnp.float32),
            pltpu.SemaphoreType.DMA,
        ],
    )
    def k(qidx_hbm, table_hbm, qrows_hbm, qidx_v, qrows_v, sem):
        wid = lax.axis_index("s") * 2 + lax.axis_index("c")
        qbase = wid * per_w
        pltpu.sync_copy(qidx_hbm.at[pl.ds(qbase, per_w)], qidx_v)
        pltpu.async_copy(table_hbm.at[qidx_v], qrows_v, sem).wait()
        pltpu.sync_copy(qrows_v, qrows_hbm.at[pl.ds(qbase, per_w)])

    return k(qidx, table)


def _support_body(ssum_ref, gcn_ref, p1_ref, p2_ref, gcnb_ref, p1b_ref,
                  p2b_ref, lna_ref, lnb_ref, sg_ref, msg_ref):
    ssum = ssum_ref[:]
    cat = jnp.concatenate([ssum[:_FEW], ssum[_FEW:]], axis=1)
    out = lax.dot_general(cat, gcn_ref[:], (((1,), (1,)), ((), ())),
                          preferred_element_type=jnp.float32)
    out = out + gcnb_ref[:] * float(_NEIGH)
    support = jnp.tanh(out * (1.0 / _FEW))
    h = lax.dot_general(support, p1_ref[:], (((1,), (1,)), ((), ())),
                        preferred_element_type=jnp.float32) + p1b_ref[:]
    h = jnp.maximum(h, 0.0)
    h2 = lax.dot_general(h, p2_ref[:], (((1,), (1,)), ((), ())),
                         preferred_element_type=jnp.float32) + p2b_ref[:]
    z = h2 + support
    mu = jnp.mean(z, axis=1, keepdims=True)
    zc = z - mu
    sigma = jnp.sqrt(jnp.sum(zc * zc, axis=1, keepdims=True) / (_D - 1.0))
    sg = lna_ref[:] * zc / (sigma + 1e-3) + lnb_ref[:]
    sg_ref[:] = sg
    msg_ref[:] = jnp.mean(sg, axis=0, keepdims=True)


def _tc_support(ssum, gcn_w_w, proj1_w, proj2_w, gcn_b2, p1b2, p2b2,
                lna2, lnb2):
    return pl.pallas_call(
        _support_body,
        out_shape=(
            jax.ShapeDtypeStruct((_FEW, _D), jnp.float32),
            jax.ShapeDtypeStruct((1, _D), jnp.float32),
        ),
    )(ssum, gcn_w_w, proj1_w, proj2_w, gcn_b2, p1b2, p2b2, lna2, lnb2)


_BLK = 4096


def _lstm_body(x0_ref, x1_ref, wih_ref, whh_ref, sg_ref, msg_ref, bg_ref,
               out_ref):
    x = jnp.concatenate([x0_ref[:], x1_ref[:]], axis=0)
    xb = x.astype(jnp.bfloat16)
    wihb = wih_ref[:].astype(jnp.bfloat16)
    whhb = whh_ref[:].astype(jnp.bfloat16)
    sg = sg_ref[:]
    sgb = sg.astype(jnp.bfloat16)
    gates_x = lax.dot_general(xb, wihb, (((1,), (1,)), ((), ())),
                              preferred_element_type=jnp.float32) + bg_ref[:]

    def attn(hq):
        logits = lax.dot_general(hq.astype(jnp.bfloat16), sgb,
                                 (((1,), (1,)), ((), ())),
                                 preferred_element_type=jnp.float32)
        m = jnp.max(logits, axis=1, keepdims=True)
        e = jnp.exp(logits - m)
        a = e / jnp.sum(e, axis=1, keepdims=True)
        r = lax.dot_general(a.astype(jnp.bfloat16), sgb,
                            (((1,), (0,)), ((), ())),
                            preferred_element_type=jnp.float32)
        return r

    g = gates_x
    c = jax.nn.sigmoid(g[:, 0:256]) * jnp.tanh(g[:, 512:768])
    h = jax.nn.sigmoid(g[:, 768:1024]) * jnp.tanh(c)
    hq = x + h[:, 0:_D]
    h_r = jnp.concatenate([hq, attn(hq)], axis=1)
    for step in range(1, 4):
        g = gates_x + lax.dot_general(h_r.astype(jnp.bfloat16), whhb,
                                      (((1,), (1,)), ((), ())),
                                      preferred_element_type=jnp.float32)
        c = (jax.nn.sigmoid(g[:, 256:512]) * c
             + jax.nn.sigmoid(g[:, 0:256]) * jnp.tanh(g[:, 512:768]))
        h = jax.nn.sigmoid(g[:, 768:1024]) * jnp.tanh(c)
        hq = x + h[:, 0:_D]
        if step < 3:
            h_r = jnp.concatenate([hq, attn(hq)], axis=1)
    hq_sum = hq[0:_BLK // 2] + hq[_BLK // 2:_BLK]
    out_ref[:] = 0.5 * lax.dot_general(msg_ref[:], hq_sum,
                                       (((1,), (1,)), ((), ())),
                                       preferred_element_type=jnp.float32)


def _tc_lstm(qrows, W_ih, W_hh, sg, msg, bg2):
    half = _BLK // 2
    return pl.pallas_call(
        _lstm_body,
        grid=(1,),
        in_specs=[
            pl.BlockSpec((half, _D), lambda i: (0, 0)),
            pl.BlockSpec((half, _D), lambda i: (1, 0)),
            pl.BlockSpec((8 * _D, _D), lambda i: (0, 0)),
            pl.BlockSpec((8 * _D, 2 * _D), lambda i: (0, 0)),
            pl.BlockSpec((_FEW, _D), lambda i: (0, 0)),
            pl.BlockSpec((1, _D), lambda i: (0, 0)),
            pl.BlockSpec((1, 8 * _D), lambda i: (0, 0)),
        ],
        out_specs=pl.BlockSpec((1, half), lambda i: (0, 0)),
        out_shape=jax.ShapeDtypeStruct((1, half), jnp.float32),
    )(qrows, qrows, W_ih, W_hh, sg, msg, bg2)


def kernel(query_pairs, support_pairs, symbol_emb, gcn_w_w, gcn_w_b,
           proj1_w, proj1_b, proj2_w, proj2_b, ln_a, ln_b,
           W_ih, W_hh, b_ih, b_hh):
    nq = _QROWS // 2
    half_q = nq // 2
    rel = support_pairs[:, :, 0]
    ent = support_pairs[:, :, 1]
    sidx = jnp.concatenate([rel, ent], axis=0).astype(jnp.int32)
    sidx = jnp.pad(sidx, ((0, 0), (0, _NPAD - _NEIGH)),
                   constant_values=_NSYM)
    sidx = sidx.reshape(-1)
    qidx_a = jnp.concatenate(
        [query_pairs[:half_q, 0], query_pairs[:half_q, 1]]).astype(jnp.int32)
    qidx_b = jnp.concatenate(
        [query_pairs[half_q:, 0], query_pairs[half_q:, 1]]).astype(jnp.int32)

    ssum = _sc_support(sidx, symbol_emb)
    qrows_a = _sc_qgather(qidx_a, symbol_emb, 2 * half_q)
    qrows_b = _sc_qgather(qidx_b, symbol_emb, 2 * half_q)

    sg, msg = _tc_support(
        ssum, gcn_w_w, proj1_w, proj2_w,
        gcn_w_b.reshape(1, _D), proj1_b.reshape(1, 2 * _D),
        proj2_b.reshape(1, _D), ln_a.reshape(1, _D), ln_b.reshape(1, _D))

    bg2 = (b_ih + b_hh).reshape(1, 8 * _D)
    s_a = _tc_lstm(qrows_a, W_ih, W_hh, sg, msg, bg2)
    s_b = _tc_lstm(qrows_b, W_ih, W_hh, sg, msg, bg2)
    return jnp.concatenate([s_a, s_b], axis=1).reshape(nq)


# speedup vs baseline: 1.0505x; 1.0505x over previous
import functools

import jax
import jax.numpy as jnp
from jax import lax
from jax.experimental import pallas as pl
from jax.experimental.pallas import tpu as pltpu
from jax.experimental.pallas import tpu_sc as plsc

_D = 128
_FEW = 64
_NEIGH = 200
_NPAD = 208
_NSYM = 100000
_QROWS = 8192
_NW = 32
_QPW = _QROWS // _NW
_SEGS = 2 * _FEW
_SPW = _SEGS // _NW


def _sc_gather(qidx, sidx, table):
    mesh = plsc.VectorSubcoreMesh(core_axis_name="c", subcore_axis_name="s")

    @functools.partial(
        pl.kernel,
        out_type=(
            jax.ShapeDtypeStruct((_QROWS, _D), jnp.float32),
            jax.ShapeDtypeStruct((_SEGS, _D), jnp.float32),
        ),
        mesh=mesh,
        scratch_types=[
            pltpu.VMEM((_QPW,), jnp.int32),
            pltpu.VMEM((2 * 64, _D), jnp.float32),
            pltpu.VMEM((_SPW, _NPAD), jnp.int32),
            pltpu.VMEM((_SPW * _NPAD, _D), jnp.float32),
            pltpu.VMEM((_SPW, _D), jnp.float32),
            pltpu.SemaphoreType.DMA,
            pltpu.SemaphoreType.DMA,
            pltpu.SemaphoreType.DMA,
            pltpu.SemaphoreType.DMA,
            pltpu.SemaphoreType.DMA,
            pltpu.SemaphoreType.DMA,
        ],
    )
    def k(qidx_hbm, sidx_hbm, table_hbm, qrows_hbm, ssum_hbm,
          qidx_v, qrows_v, sidx_v, srows_v, acc_v, qsem, qosem, s0, s1, s2, s3):
        wid = lax.axis_index("s") * 2 + lax.axis_index("c")
        qbase = wid * _QPW
        sbase = wid * _SPW
        ssems = [s0, s1, s2, s3]
        pltpu.sync_copy(qidx_hbm.at[pl.ds(qbase, _QPW)], qidx_v)
        pltpu.sync_copy(sidx_hbm.at[pl.ds(sbase, _SPW)], sidx_v)

        s_cp = []
        for t in range(_SPW):
            base = t * _NPAD
            c0 = pltpu.async_copy(
                table_hbm.at[sidx_v.at[t, pl.ds(0, 128)]],
                srows_v.at[pl.ds(base, 128)], ssems[t])
            c1 = pltpu.async_copy(
                table_hbm.at[sidx_v.at[t, pl.ds(128, _NPAD - 128)]],
                srows_v.at[pl.ds(base + 128, _NPAD - 128)], ssems[t])
            s_cp.append((c0, c1))

        def q_fire(ch, slot):
            return pltpu.async_copy(
                table_hbm.at[qidx_v.at[pl.ds(ch * 64, 64)]],
                qrows_v.at[pl.ds(slot * 64, 64)], qsem)

        def q_flush(ch, slot):
            return pltpu.async_copy(
                qrows_v.at[pl.ds(slot * 64, 64)],
                qrows_hbm.at[pl.ds(qbase + ch * 64, 64)], qosem)

        q_cp = [q_fire(0, 0), q_fire(1, 1)]

        def accumulate(t):
            base = t * _NPAD

            def body(r, accs):
                new = accs
                for rr in range(4):
                    row = base + r * 4 + rr
                    new = tuple(a + srows_v[row, pl.ds(kk * 16, 16)]
                                for kk, a in enumerate(new))
                return new

            accs = lax.fori_loop(
                0, _NPAD // 4, body,
                tuple(jnp.zeros((16,), jnp.float32) for _ in range(_D // 16)))
            for kk in range(_D // 16):
                acc_v[t, pl.ds(kk * 16, 16)] = accs[kk]

        q_cp[0].wait()
        o0 = q_flush(0, 0)
        q_cp[1].wait()
        o1 = q_flush(1, 1)
        o0.wait()
        c2 = q_fire(2, 0)
        o1.wait()
        c3 = q_fire(3, 1)
        c2.wait()
        o2 = q_flush(2, 0)
        c3.wait()
        o3 = q_flush(3, 1)
        for t in range(_SPW):
            s_cp[t][0].wait()
            s_cp[t][1].wait()
            accumulate(t)
        pltpu.sync_copy(acc_v, ssum_hbm.at[pl.ds(sbase, _SPW)])
        o2.wait()
        o3.wait()

    return k(qidx, sidx, table)


_BLK = 4096


def _lstm_body(x0_ref, x1_ref, wih_ref, whh_ref, ssum_ref, gcn_ref, p1_ref,
               p2_ref, gcnb_ref, p1b_ref, p2b_ref, lna_ref, lnb_ref, bg_ref,
               out_ref):
    ssum = ssum_ref[:]
    cat = jnp.concatenate([ssum[:_FEW], ssum[_FEW:]], axis=1)
    out = lax.dot_general(cat, gcn_ref[:], (((1,), (1,)), ((), ())),
                          preferred_element_type=jnp.float32)
    out = out + gcnb_ref[:] * float(_NEIGH)
    support = jnp.tanh(out * (1.0 / _FEW))
    hs = lax.dot_general(support, p1_ref[:], (((1,), (1,)), ((), ())),
                         preferred_element_type=jnp.float32) + p1b_ref[:]
    hs = jnp.maximum(hs, 0.0)
    h2 = lax.dot_general(hs, p2_ref[:], (((1,), (1,)), ((), ())),
                         preferred_element_type=jnp.float32) + p2b_ref[:]
    z = h2 + support
    mu = jnp.mean(z, axis=1, keepdims=True)
    zc = z - mu
    sigma = jnp.sqrt(jnp.sum(zc * zc, axis=1, keepdims=True) / (_D - 1.0))
    sg = lna_ref[:] * zc / (sigma + 1e-3) + lnb_ref[:]
    msg = jnp.mean(sg, axis=0, keepdims=True)
    x = jnp.concatenate([x0_ref[:], x1_ref[:]], axis=0)
    xb = x.astype(jnp.bfloat16)
    wihb = wih_ref[:].astype(jnp.bfloat16)
    whhb = whh_ref[:].astype(jnp.bfloat16)
    sgb = sg.astype(jnp.bfloat16)
    gates_x = lax.dot_general(xb, wihb, (((1,), (1,)), ((), ())),
                              preferred_element_type=jnp.float32) + bg_ref[:]

    def attn(hq):
        logits = lax.dot_general(hq.astype(jnp.bfloat16), sgb,
                                 (((1,), (1,)), ((), ())),
                                 preferred_element_type=jnp.float32)
        m = jnp.max(logits, axis=1, keepdims=True)
        e = jnp.exp(logits - m)
        a = e / jnp.sum(e, axis=1, keepdims=True)
        r = lax.dot_general(a.astype(jnp.bfloat16), sgb,
                            (((1,), (0,)), ((), ())),
                            preferred_element_type=jnp.float32)
        return r

    g = gates_x
    c = jax.nn.sigmoid(g[:, 0:256]) * jnp.tanh(g[:, 512:768])
    h = jax.nn.sigmoid(g[:, 768:1024]) * jnp.tanh(c)
    hq = x + h[:, 0:_D]
    h_r = jnp.concatenate([hq, attn(hq)], axis=1)
    for step in range(1, 4):
        g = gates_x + lax.dot_general(h_r.astype(jnp.bfloat16), whhb,
                                      (((1,), (1,)), ((), ())),
                                      preferred_element_type=jnp.float32)
        c = (jax.nn.sigmoid(g[:, 256:512]) * c
             + jax.nn.sigmoid(g[:, 0:256]) * jnp.tanh(g[:, 512:768]))
        h = jax.nn.sigmoid(g[:, 768:1024]) * jnp.tanh(c)
        hq = x + h[:, 0:_D]
        if step < 3:
            h_r = jnp.concatenate([hq, attn(hq)], axis=1)
    hq_sum = hq[0:_BLK // 2] + hq[_BLK // 2:_BLK]
    out_ref[:] = 0.5 * lax.dot_general(msg, hq_sum,
                                       (((1,), (1,)), ((), ())),
                                       preferred_element_type=jnp.float32)


def _tc_lstm(qrows, W_ih, W_hh, ssum, gcn_w_w, proj1_w, proj2_w,
             gcn_b2, p1b2, p2b2, lna2, lnb2, bg2):
    nblk = _QROWS // _BLK
    half = _BLK // 2
    full = lambda i: (0, 0)
    return pl.pallas_call(
        _lstm_body,
        grid=(nblk,),
        in_specs=[
            pl.BlockSpec((half, _D), lambda i: (i, 0)),
            pl.BlockSpec((half, _D), lambda i: (i + _QROWS // 2 // (_BLK // 2), 0)),
            pl.BlockSpec((8 * _D, _D), full),
            pl.BlockSpec((8 * _D, 2 * _D), full),
            pl.BlockSpec((_SEGS, _D), full),
            pl.BlockSpec((_D, 2 * _D), full),
            pl.BlockSpec((2 * _D, _D), full),
            pl.BlockSpec((_D, 2 * _D), full),
            pl.BlockSpec((1, _D), full),
            pl.BlockSpec((1, 2 * _D), full),
            pl.BlockSpec((1, _D), full),
            pl.BlockSpec((1, _D), full),
            pl.BlockSpec((1, _D), full),
            pl.BlockSpec((1, 8 * _D), full),
        ],
        out_specs=pl.BlockSpec((1, half), lambda i: (0, i)),
        out_shape=jax.ShapeDtypeStruct((1, _QROWS // 2), jnp.float32),
    )(qrows, qrows, W_ih, W_hh, ssum, gcn_w_w, proj1_w, proj2_w,
      gcn_b2, p1b2, p2b2, lna2, lnb2, bg2)


def kernel(query_pairs, support_pairs, symbol_emb, gcn_w_w, gcn_w_b,
           proj1_w, proj1_b, proj2_w, proj2_b, ln_a, ln_b,
           W_ih, W_hh, b_ih, b_hh):
    qidx = jnp.concatenate(
        [query_pairs[:, 0], query_pairs[:, 1]]).astype(jnp.int32)
    rel = support_pairs[:, :, 0]
    ent = support_pairs[:, :, 1]
    sidx = jnp.concatenate([rel, ent], axis=0).astype(jnp.int32)
    sidx = jnp.pad(sidx, ((0, 0), (0, _NPAD - _NEIGH)),
                   constant_values=_NSYM)

    qrows, ssum = _sc_gather(qidx, sidx, symbol_emb)

    bg2 = (b_ih + b_hh).reshape(1, 8 * _D)
    s = _tc_lstm(
        qrows, W_ih, W_hh, ssum, gcn_w_w, proj1_w, proj2_w,
        gcn_w_b.reshape(1, _D), proj1_b.reshape(1, 2 * _D),
        proj2_b.reshape(1, _D), ln_a.reshape(1, _D), ln_b.reshape(1, _D),
        bg2)
    return s.reshape(_QROWS // 2)
